# Initial kernel scaffold; baseline (speedup 1.0000x reference)
import jax
import jax.numpy as jnp
from jax.experimental import pallas as pl


def kernel(x, edge_index, edge_weights, Wl1, bl1, Wr1, br1, Wl2, bl2, Wr2, br2, W3, b3):
    raise NotImplementedError("write your pallas kernel here")


# speedup vs baseline: 5.4677x; 5.4677x over previous
import functools

import jax
import jax.numpy as jnp
from jax import lax
from jax.experimental import pallas as pl
from jax.experimental.pallas import tpu as pltpu
from jax.experimental.pallas import tpu_sc as plsc

_NC = 2
_NS = 16
_NPAD = 10240
_HP = 32
_EPAD = 163840
_EPT = _EPAD // (_NC * _NS)
_CHUNK = 128
_ZR = 128
_MBLK = 1024


def _proj_body(x_ref, wl_ref, wr_ref, b_ref, y_ref, r_ref):
    xb = x_ref[...]
    y_ref[...] = jnp.dot(xb, wl_ref[...], preferred_element_type=jnp.float32)
    r_ref[...] = jnp.dot(xb, wr_ref[...], preferred_element_type=jnp.float32) + b_ref[...]


def _mid_body(p_ref, r_ref, wl2_ref, h_ref, y2_ref):
    h = jnp.maximum(p_ref[0] + p_ref[1] + r_ref[...], 0.0)
    h_ref[...] = h
    y2_ref[...] = jnp.dot(h, wl2_ref[...], preferred_element_type=jnp.float32)


def _out_body(p_ref, h_ref, wr2_ref, b2_ref, w3_ref, b3_ref, o_ref):
    h2 = jnp.maximum(
        p_ref[0] + p_ref[1]
        + jnp.dot(h_ref[...], wr2_ref[...], preferred_element_type=jnp.float32)
        + b2_ref[...],
        0.0,
    )
    o_ref[...] = jnp.dot(h2, w3_ref[...], preferred_element_type=jnp.float32) + b3_ref[...]


def _sc_scatter(y, src, dst, wts):
    mesh = plsc.VectorSubcoreMesh(core_axis_name="c", subcore_axis_name="s")

    @functools.partial(
        pl.kernel,
        mesh=mesh,
        out_type=jax.ShapeDtypeStruct((_NC * _NPAD, _HP), jnp.float32),
        compiler_params=pltpu.CompilerParams(use_tc_tiling_on_sc=False),
        scratch_types=[
            pltpu.VMEM((_CHUNK,), jnp.int32),
            pltpu.VMEM((_CHUNK,), jnp.int32),
            pltpu.VMEM((_CHUNK,), jnp.float32),
            pltpu.VMEM((_CHUNK, _HP), jnp.float32),
            pltpu.VMEM((_CHUNK, _HP), jnp.float32),
            pltpu.VMEM((_ZR, _HP), jnp.float32),
            pltpu.VMEM_SHARED((_NPAD, _HP), jnp.float32),
            pltpu.SemaphoreType.DMA,
        ],
    )
    def run(y_hbm, src_hbm, dst_hbm, w_hbm, out_hbm,
            src_v, dst_v, w_v, grows_v, msg_v, z_v, acc_sh, sem):
        c = lax.axis_index("c")
        s = lax.axis_index("s")
        tid = c * _NS + s
        rows_per_tile = _NPAD // _NS

        z16 = jnp.zeros((16,), jnp.float32)
        for i in range(_ZR):
            z_v[i, 0:16] = z16
            z_v[i, 16:32] = z16

        for j in range(rows_per_tile // _ZR):
            pltpu.async_copy(
                z_v, acc_sh.at[pl.ds(s * rows_per_tile + j * _ZR, _ZR)], sem
            ).wait()
        plsc.subcore_barrier()

        @pl.loop(0, _EPT // _CHUNK)
        def chunk(j):
            e0 = tid * _EPT + j * _CHUNK
            pltpu.async_copy(src_hbm.at[pl.ds(e0, _CHUNK)], src_v, sem).wait()
            pltpu.async_copy(dst_hbm.at[pl.ds(e0, _CHUNK)], dst_v, sem).wait()
            pltpu.async_copy(w_hbm.at[pl.ds(e0, _CHUNK)], w_v, sem).wait()
            pltpu.async_copy(y_hbm.at[src_v], grows_v, sem).wait()
            for g in range(_CHUNK // 16):
                wv = w_v[pl.ds(g * 16, 16)]
                for jj in range(16):
                    w = wv[jj]
                    i = g * 16 + jj
                    msg_v[i, 0:16] = grows_v[i, 0:16] * w
                    msg_v[i, 16:32] = grows_v[i, 16:32] * w
            add_desc = pltpu.make_async_copy(msg_v, acc_sh.at[dst_v], sem)
            add_desc.start(add=True)
            add_desc.wait()

        plsc.subcore_barrier()

        for j in range(rows_per_tile // _ZR):
            r0 = s * rows_per_tile + j * _ZR
            pltpu.async_copy(acc_sh.at[pl.ds(r0, _ZR)], z_v, sem).wait()
            pltpu.async_copy(
                z_v, out_hbm.at[pl.ds(c * _NPAD + r0, _ZR)], sem
            ).wait()

    return run(y, src, dst, wts).reshape(_NC, _NPAD, _HP)


def kernel(x, edge_index, edge_weights, Wl1, bl1, Wr1, br1, Wl2, bl2, Wr2, br2, W3, b3):
    N, D = x.shape
    H = Wl1.shape[1]
    C = W3.shape[1]
    E = edge_index.shape[1]

    x_p = jnp.zeros((_NPAD, D), jnp.float32).at[:N].set(x)
    src = jnp.zeros((_EPAD,), jnp.int32).at[:E].set(edge_index[0])
    dst = jnp.zeros((_EPAD,), jnp.int32).at[:E].set(edge_index[1])
    wts = jnp.zeros((_EPAD,), jnp.float32).at[:E].set(edge_weights)

    Wl1p = jnp.zeros((D, _HP), jnp.float32).at[:, :H].set(Wl1)
    Wr1p = jnp.zeros((D, _HP), jnp.float32).at[:, :H].set(Wr1)
    b1p = jnp.zeros((1, _HP), jnp.float32).at[0, :H].set(bl1 + br1)
    Wl2p = jnp.zeros((_HP, _HP), jnp.float32).at[:H, :H].set(Wl2)
    Wr2p = jnp.zeros((_HP, _HP), jnp.float32).at[:H, :H].set(Wr2)
    b2p = jnp.zeros((1, _HP), jnp.float32).at[0, :H].set(bl2 + br2)
    W3p = jnp.zeros((_HP, C), jnp.float32).at[:H].set(W3)
    b3p = b3[None, :]

    grid = _NPAD // _MBLK

    y1, r1 = pl.pallas_call(
        _proj_body,
        grid=(grid,),
        in_specs=[
            pl.BlockSpec((_MBLK, D), lambda i: (i, 0)),
            pl.BlockSpec((D, _HP), lambda i: (0, 0)),
            pl.BlockSpec((D, _HP), lambda i: (0, 0)),
            pl.BlockSpec((1, _HP), lambda i: (0, 0)),
        ],
        out_specs=[
            pl.BlockSpec((_MBLK, _HP), lambda i: (i, 0)),
            pl.BlockSpec((_MBLK, _HP), lambda i: (i, 0)),
        ],
        out_shape=[
            jax.ShapeDtypeStruct((_NPAD, _HP), jnp.float32),
            jax.ShapeDtypeStruct((_NPAD, _HP), jnp.float32),
        ],
    )(x_p, Wl1p, Wr1p, b1p)

    part1 = _sc_scatter(y1, src, dst, wts)

    h1, y2 = pl.pallas_call(
        _mid_body,
        grid=(grid,),
        in_specs=[
            pl.BlockSpec((_NC, _MBLK, _HP), lambda i: (0, i, 0)),
            pl.BlockSpec((_MBLK, _HP), lambda i: (i, 0)),
            pl.BlockSpec((_HP, _HP), lambda i: (0, 0)),
        ],
        out_specs=[
            pl.BlockSpec((_MBLK, _HP), lambda i: (i, 0)),
            pl.BlockSpec((_MBLK, _HP), lambda i: (i, 0)),
        ],
        out_shape=[
            jax.ShapeDtypeStruct((_NPAD, _HP), jnp.float32),
            jax.ShapeDtypeStruct((_NPAD, _HP), jnp.float32),
        ],
    )(part1, r1, Wl2p)

    part2 = _sc_scatter(y2, src, dst, wts)

    out_p = pl.pallas_call(
        _out_body,
        grid=(grid,),
        in_specs=[
            pl.BlockSpec((_NC, _MBLK, _HP), lambda i: (0, i, 0)),
            pl.BlockSpec((_MBLK, _HP), lambda i: (i, 0)),
            pl.BlockSpec((_HP, _HP), lambda i: (0, 0)),
            pl.BlockSpec((1, _HP), lambda i: (0, 0)),
            pl.BlockSpec((_HP, C), lambda i: (0, 0)),
            pl.BlockSpec((1, C), lambda i: (0, 0)),
        ],
        out_specs=pl.BlockSpec((_MBLK, C), lambda i: (i, 0)),
        out_shape=jax.ShapeDtypeStruct((_NPAD, C), jnp.float32),
    )(part2, h1, Wr2p, b2p, W3p, b3p)

    return out_p[:N]


# speedup vs baseline: 8.1597x; 1.4924x over previous
import functools

import jax
import jax.numpy as jnp
from jax import lax
from jax.experimental import pallas as pl
from jax.experimental.pallas import tpu as pltpu
from jax.experimental.pallas import tpu_sc as plsc

_NC = 2
_NS = 16
_NPAD = 10240
_HP = 32
_EPAD = 163840
_EPT = _EPAD // (_NC * _NS)
_CHUNK = 128
_GRP = 8
_ZR = 128
_MBLK = 1024


def _proj_body(x_ref, wl_ref, wr_ref, b_ref, y_ref, r_ref):
    xb = x_ref[...]
    y_ref[...] = jnp.dot(xb, wl_ref[...], preferred_element_type=jnp.float32)
    r_ref[...] = jnp.dot(xb, wr_ref[...], preferred_element_type=jnp.float32) + b_ref[...]


def _mid_body(p_ref, r_ref, wl2_ref, h_ref, y2_ref):
    h = jnp.maximum(p_ref[0] + p_ref[1] + r_ref[...], 0.0)
    h_ref[...] = h
    y2_ref[...] = jnp.dot(h, wl2_ref[...], preferred_element_type=jnp.float32)


def _out_body(p_ref, h_ref, wr2_ref, b2_ref, w3_ref, b3_ref, o_ref):
    h2 = jnp.maximum(
        p_ref[0] + p_ref[1]
        + jnp.dot(h_ref[...], wr2_ref[...], preferred_element_type=jnp.float32)
        + b2_ref[...],
        0.0,
    )
    o_ref[...] = jnp.dot(h2, w3_ref[...], preferred_element_type=jnp.float32) + b3_ref[...]


def _sc_scatter(y, src, dst, wts):
    mesh = plsc.VectorSubcoreMesh(core_axis_name="c", subcore_axis_name="s")

    @functools.partial(
        pl.kernel,
        mesh=mesh,
        out_type=jax.ShapeDtypeStruct((_NC * _NPAD, _HP), jnp.float32),
        compiler_params=pltpu.CompilerParams(use_tc_tiling_on_sc=False),
        scratch_types=[
            pltpu.VMEM((_EPT,), jnp.int32),
            pltpu.VMEM((_EPT // _CHUNK, _CHUNK), jnp.int32),
            pltpu.VMEM((_EPT,), jnp.float32),
            pltpu.VMEM((_GRP * _CHUNK, _HP), jnp.float32),
            pltpu.VMEM((_ZR, _HP), jnp.float32),
            pltpu.VMEM_SHARED((_NPAD, _HP), jnp.float32),
            pltpu.SemaphoreType.DMA,
            pltpu.SemaphoreType.DMA,
        ],
    )
    def run(y_hbm, src_hbm, dst_hbm, w_hbm, out_hbm,
            src_v, dst_v, w_v, rows_v, z_v, acc_sh, gsem, ssem):
        c = lax.axis_index("c")
        s = lax.axis_index("s")
        tid = c * _NS + s
        rows_per_tile = _NPAD // _NS
        nchunks = _EPT // _CHUNK

        pltpu.async_copy(src_hbm.at[pl.ds(tid * _EPT, _EPT)], src_v, gsem).wait()
        pltpu.async_copy(
            dst_hbm.at[pl.ds(tid * nchunks, nchunks)], dst_v, gsem
        ).wait()
        pltpu.async_copy(w_hbm.at[pl.ds(tid * _EPT, _EPT)], w_v, gsem).wait()

        z16 = jnp.zeros((16,), jnp.float32)
        for i in range(_ZR):
            z_v[i, 0:16] = z16
            z_v[i, 16:32] = z16

        for j in range(rows_per_tile // _ZR):
            pltpu.async_copy(
                z_v, acc_sh.at[pl.ds(s * rows_per_tile + j * _ZR, _ZR)], ssem
            ).wait()
        plsc.subcore_barrier()

        @pl.loop(0, nchunks // _GRP)
        def group(j):
            c0 = j * _GRP
            for b in range(_GRP):
                pltpu.async_copy(
                    y_hbm.at[src_v.at[pl.ds((c0 + b) * _CHUNK, _CHUNK)]],
                    rows_v.at[pl.ds(b * _CHUNK, _CHUNK)],
                    gsem,
                )
            for b in range(_GRP):
                pltpu.make_async_copy(
                    y_hbm.at[src_v.at[pl.ds((c0 + b) * _CHUNK, _CHUNK)]],
                    rows_v.at[pl.ds(b * _CHUNK, _CHUNK)],
                    gsem,
                ).wait()

            @pl.loop(0, _GRP * _CHUNK // 16)
            def scale(g):
                wv = w_v[pl.ds(c0 * _CHUNK + g * 16, 16)]
                for jj in range(16):
                    w = wv[jj]
                    rows_v[g * 16 + jj, 0:16] = rows_v[g * 16 + jj, 0:16] * w
                    rows_v[g * 16 + jj, 16:32] = rows_v[g * 16 + jj, 16:32] * w

            for b in range(_GRP):
                pltpu.async_copy(
                    rows_v.at[pl.ds(b * _CHUNK, _CHUNK)],
                    acc_sh.at[dst_v.at[c0 + b]],
                    ssem,
                    add=True,
                )
            for b in range(_GRP):
                pltpu.make_async_copy(
                    rows_v.at[pl.ds(b * _CHUNK, _CHUNK)],
                    acc_sh.at[dst_v.at[c0 + b]],
                    ssem,
                ).wait()

        plsc.subcore_barrier()

        for j in range(rows_per_tile // _ZR):
            r0 = s * rows_per_tile + j * _ZR
            pltpu.async_copy(acc_sh.at[pl.ds(r0, _ZR)], z_v, gsem).wait()
            pltpu.async_copy(
                z_v, out_hbm.at[pl.ds(c * _NPAD + r0, _ZR)], gsem
            ).wait()

    return run(y, src, dst, wts).reshape(_NC, _NPAD, _HP)


def kernel(x, edge_index, edge_weights, Wl1, bl1, Wr1, br1, Wl2, bl2, Wr2, br2, W3, b3):
    N, D = x.shape
    H = Wl1.shape[1]
    C = W3.shape[1]
    E = edge_index.shape[1]

    x_p = jnp.zeros((_NPAD, D), jnp.float32).at[:N].set(x)
    src = jnp.zeros((_EPAD,), jnp.int32).at[:E].set(edge_index[0])
    dst = jnp.zeros((_EPAD,), jnp.int32).at[:E].set(edge_index[1]).reshape(
        _EPAD // _CHUNK, _CHUNK)
    wts = jnp.zeros((_EPAD,), jnp.float32).at[:E].set(edge_weights)

    Wl1p = jnp.zeros((D, _HP), jnp.float32).at[:, :H].set(Wl1)
    Wr1p = jnp.zeros((D, _HP), jnp.float32).at[:, :H].set(Wr1)
    b1p = jnp.zeros((1, _HP), jnp.float32).at[0, :H].set(bl1 + br1)
    Wl2p = jnp.zeros((_HP, _HP), jnp.float32).at[:H, :H].set(Wl2)
    Wr2p = jnp.zeros((_HP, _HP), jnp.float32).at[:H, :H].set(Wr2)
    b2p = jnp.zeros((1, _HP), jnp.float32).at[0, :H].set(bl2 + br2)
    W3p = jnp.zeros((_HP, C), jnp.float32).at[:H].set(W3)
    b3p = b3[None, :]

    grid = _NPAD // _MBLK

    y1, r1 = pl.pallas_call(
        _proj_body,
        grid=(grid,),
        in_specs=[
            pl.BlockSpec((_MBLK, D), lambda i: (i, 0)),
            pl.BlockSpec((D, _HP), lambda i: (0, 0)),
            pl.BlockSpec((D, _HP), lambda i: (0, 0)),
            pl.BlockSpec((1, _HP), lambda i: (0, 0)),
        ],
        out_specs=[
            pl.BlockSpec((_MBLK, _HP), lambda i: (i, 0)),
            pl.BlockSpec((_MBLK, _HP), lambda i: (i, 0)),
        ],
        out_shape=[
            jax.ShapeDtypeStruct((_NPAD, _HP), jnp.float32),
            jax.ShapeDtypeStruct((_NPAD, _HP), jnp.float32),
        ],
    )(x_p, Wl1p, Wr1p, b1p)

    part1 = _sc_scatter(y1, src, dst, wts)

    h1, y2 = pl.pallas_call(
        _mid_body,
        grid=(grid,),
        in_specs=[
            pl.BlockSpec((_NC, _MBLK, _HP), lambda i: (0, i, 0)),
            pl.BlockSpec((_MBLK, _HP), lambda i: (i, 0)),
            pl.BlockSpec((_HP, _HP), lambda i: (0, 0)),
        ],
        out_specs=[
            pl.BlockSpec((_MBLK, _HP), lambda i: (i, 0)),
            pl.BlockSpec((_MBLK, _HP), lambda i: (i, 0)),
        ],
        out_shape=[
            jax.ShapeDtypeStruct((_NPAD, _HP), jnp.float32),
            jax.ShapeDtypeStruct((_NPAD, _HP), jnp.float32),
        ],
    )(part1, r1, Wl2p)

    part2 = _sc_scatter(y2, src, dst, wts)

    out_p = pl.pallas_call(
        _out_body,
        grid=(grid,),
        in_specs=[
            pl.BlockSpec((_NC, _MBLK, _HP), lambda i: (0, i, 0)),
            pl.BlockSpec((_MBLK, _HP), lambda i: (i, 0)),
            pl.BlockSpec((_HP, _HP), lambda i: (0, 0)),
            pl.BlockSpec((1, _HP), lambda i: (0, 0)),
            pl.BlockSpec((_HP, C), lambda i: (0, 0)),
            pl.BlockSpec((1, C), lambda i: (0, 0)),
        ],
        out_specs=pl.BlockSpec((_MBLK, C), lambda i: (i, 0)),
        out_shape=jax.ShapeDtypeStruct((_NPAD, C), jnp.float32),
    )(part2, h1, Wr2p, b2p, W3p, b3p)

    return out_p[:N]
